# Initial kernel scaffold; baseline (speedup 1.0000x reference)
import jax
import jax.numpy as jnp
from jax.experimental import pallas as pl


def kernel(graph_data, grid_data, lat_lon_coords, graph_time_indices, grid_time_indices, Wg, bg, Wr, br, bn_weight, bn_bias):
    raise NotImplementedError("write your pallas kernel here")


# speedup vs baseline: 2.8282x; 2.8282x over previous
import functools

import jax
import jax.numpy as jnp
from jax import lax
from jax.experimental import pallas as pl
from jax.experimental.pallas import tpu as pltpu
from jax.experimental.pallas import tpu_sc as plsc

B = 2
T = 12
N = 50000
C = 32
HW = 4096
FD = 64
GRID_CH = 16

NSUB = 16
CHUNK = 112
NCH = (N + CHUNK - 1) // CHUNK
TAIL = N - (NCH - 1) * CHUNK
CH_PER_SUB = (NCH + NSUB - 1) // NSUB
PADN = NCH * CHUNK
NPH = 3
TPH = T // NPH
RW = TPH * C
ROWS_PER_SUB = HW // NSUB
ZROWS = 64
DUMMY = HW
BIG = 1 << 27


def _sc_scatter(lat_pad, lon_pad, graph_ph):
    mesh = plsc.VectorSubcoreMesh(core_axis_name="c", subcore_axis_name="s")

    @functools.partial(
        pl.kernel,
        mesh=mesh,
        out_type=jax.ShapeDtypeStruct((B * NPH * HW, RW), jnp.float32),
        scratch_types=[
            pltpu.VMEM((ZROWS, RW), jnp.float32),
            pltpu.VMEM((CHUNK,), jnp.float32),
            pltpu.VMEM((CHUNK,), jnp.float32),
            pltpu.VMEM((CH_PER_SUB, 128), jnp.int32),
            pltpu.VMEM((128, RW), jnp.float32),
            pltpu.VMEM_SHARED((HW + 8, RW), jnp.float32),
        ],
    )
    def scatter_kernel(lat_hbm, lon_hbm, graph_hbm, out_hbm,
                       zbuf, latv, lonv, idxv, rowsv, accum):
        c = lax.axis_index("c")
        s = lax.axis_index("s")
        iota16 = lax.iota(jnp.int32, 16)
        zeros16 = jnp.zeros((16,), jnp.float32)
        dummy16 = jnp.full((16,), DUMMY, jnp.int32)

        def _zrow(i, carry):
            def _z16(k, carry2):
                zbuf[i, pl.ds(k * 16, 16)] = zeros16
                return carry2
            lax.fori_loop(0, RW // 16, _z16, 0)
            return carry
        lax.fori_loop(0, ZROWS, _zrow, 0)

        def _zero_slice():
            def _zcopy(r, carry):
                pltpu.sync_copy(
                    zbuf,
                    accum.at[pl.ds(s * ROWS_PER_SUB + r * ZROWS, ZROWS)])
                return carry
            lax.fori_loop(0, ROWS_PER_SUB // ZROWS, _zcopy, 0)

        _zero_slice()

        def _cells(jj, carry):
            j = jj * NSUB + s

            @pl.when(j < NCH)
            def _():
                pltpu.sync_copy(lat_hbm.at[pl.ds(c * PADN + j * CHUNK, CHUNK)],
                                latv)
                pltpu.sync_copy(lon_hbm.at[pl.ds(c * PADN + j * CHUNK, CHUNK)],
                                lonv)

                def _grp(k, carry2):
                    la = latv[pl.ds(k * 16, 16)]
                    lo = lonv[pl.ds(k * 16, 16)]
                    li = jnp.clip((la * 64.0).astype(jnp.int32), 0, 63)
                    loi = jnp.clip((lo * 64.0).astype(jnp.int32), 0, 63)
                    cell = li * 64 + loi
                    pos = j * CHUNK + k * 16 + iota16
                    cell = jnp.where(pos < N, cell, BIG)
                    idxv[jj, pl.ds(k * 16, 16)] = jnp.minimum(cell, DUMMY)
                    return carry2
                lax.fori_loop(0, CHUNK // 16, _grp, 0)
                idxv[jj, pl.ds(CHUNK, 16)] = dummy16
            return carry
        lax.fori_loop(0, CH_PER_SUB, _cells, 0)
        plsc.subcore_barrier()

        def _phase(p, carry):
            def _chunk(jj, carry2):
                j = jj * NSUB + s

                @pl.when(j < NCH - 1)
                def _():
                    pltpu.sync_copy(
                        graph_hbm.at[c, p, pl.ds(j * CHUNK, CHUNK)],
                        rowsv.at[pl.ds(0, CHUNK)])

                @pl.when(j == NCH - 1)
                def _():
                    pltpu.sync_copy(
                        graph_hbm.at[c, p, pl.ds((NCH - 1) * CHUNK, TAIL)],
                        rowsv.at[pl.ds(0, TAIL)])

                @pl.when(j < NCH)
                def _():
                    pltpu.sync_copy(rowsv, accum.at[idxv.at[jj]], add=True)
                return carry2
            lax.fori_loop(0, CH_PER_SUB, _chunk, 0)

            plsc.subcore_barrier()
            pltpu.sync_copy(
                accum.at[pl.ds(s * ROWS_PER_SUB, ROWS_PER_SUB)],
                out_hbm.at[pl.ds((c * NPH + p) * HW + s * ROWS_PER_SUB,
                                 ROWS_PER_SUB)])
            _zero_slice()
            plsc.subcore_barrier()
            return carry
        lax.fori_loop(0, NPH, _phase, 0)

    return scatter_kernel(lat_pad, lon_pad, graph_ph)


def _proj(interp128, grid64, tt, wg_ref, bg_ref, wr_ref, br_ref):
    interp = interp128[:, tt * C:(tt + 1) * C]
    grid = grid64[tt * GRID_CH:(tt + 1) * GRID_CH, :]
    y1 = lax.dot_general(wg_ref[...], interp, (((1,), (1,)), ((), ())),
                         preferred_element_type=jnp.float32)
    y1 = y1 + jnp.reshape(bg_ref[...], (FD // 2, 1))
    y2 = lax.dot_general(wr_ref[...], grid, (((1,), (0,)), ((), ())),
                         preferred_element_type=jnp.float32)
    y2 = y2 + jnp.reshape(br_ref[...], (FD // 2, 1))
    return y1, y2


def _stats_body(interp_ref, grid_ref, wg_ref, bg_ref, wr_ref, br_ref,
                out_ref, acc):
    i = pl.program_id(0)

    @pl.when(i == 0)
    def _():
        acc[...] = jnp.zeros_like(acc)

    interp128 = interp_ref[0, 0]
    grid64 = grid_ref[0]
    for tt in range(TPH):
        y1, y2 = _proj(interp128, grid64, tt, wg_ref, bg_ref, wr_ref, br_ref)
        s1 = jnp.reshape(jnp.sum(y1, axis=1), (1, FD // 2))
        q1 = jnp.reshape(jnp.sum(y1 * y1, axis=1), (1, FD // 2))
        s2 = jnp.reshape(jnp.sum(y2, axis=1), (1, FD // 2))
        q2 = jnp.reshape(jnp.sum(y2 * y2, axis=1), (1, FD // 2))
        sums = jnp.concatenate([s1, s2], axis=1)
        sqs = jnp.concatenate([q1, q2], axis=1)
        acc[...] += jnp.concatenate([sums, sqs], axis=0)
    out_ref[...] = acc[...]


def _stats_call(interp4, grid6, wg, bg2, wr, br2, interpret=False):
    return pl.pallas_call(
        _stats_body,
        grid=(B * NPH,),
        in_specs=[
            pl.BlockSpec((1, 1, HW, RW), lambda i: (i // NPH, i % NPH, 0, 0)),
            pl.BlockSpec((1, TPH * GRID_CH, HW), lambda i: (i, 0, 0)),
            pl.BlockSpec((FD // 2, C), lambda i: (0, 0)),
            pl.BlockSpec((1, FD // 2), lambda i: (0, 0)),
            pl.BlockSpec((FD // 2, GRID_CH), lambda i: (0, 0)),
            pl.BlockSpec((1, FD // 2), lambda i: (0, 0)),
        ],
        out_specs=pl.BlockSpec((2, FD), lambda i: (0, 0)),
        out_shape=jax.ShapeDtypeStruct((2, FD), jnp.float32),
        scratch_shapes=[pltpu.VMEM((2, FD), jnp.float32)],
        interpret=interpret,
    )(interp4, grid6, wg, bg2, wr, br2)


def _norm_body(stats_ref, interp_ref, grid_ref, wg_ref, bg_ref, wr_ref,
               br_ref, bnw_ref, bnb_ref, out_ref):
    pcount = float(B * T * HW)
    st = stats_ref[...]
    mean = st[0:1, :] / pcount
    var = st[1:2, :] / pcount - mean * mean
    scale = bnw_ref[...] * lax.rsqrt(var + 1e-5)
    shift = bnb_ref[...] - mean * scale
    scale_c = jnp.reshape(scale, (FD, 1))
    shift_c = jnp.reshape(shift, (FD, 1))
    interp128 = interp_ref[0, 0]
    grid64 = grid_ref[0]
    for tt in range(TPH):
        y1, y2 = _proj(interp128, grid64, tt, wg_ref, bg_ref, wr_ref, br_ref)
        y = jnp.concatenate([y1, y2], axis=0)
        out_ref[tt] = y * scale_c + shift_c


def _norm_call(stats, interp4, grid6, wg, bg2, wr, br2, bnw2, bnb2,
               interpret=False):
    return pl.pallas_call(
        _norm_body,
        grid=(B * NPH,),
        in_specs=[
            pl.BlockSpec((2, FD), lambda i: (0, 0)),
            pl.BlockSpec((1, 1, HW, RW), lambda i: (i // NPH, i % NPH, 0, 0)),
            pl.BlockSpec((1, TPH * GRID_CH, HW), lambda i: (i, 0, 0)),
            pl.BlockSpec((FD // 2, C), lambda i: (0, 0)),
            pl.BlockSpec((1, FD // 2), lambda i: (0, 0)),
            pl.BlockSpec((FD // 2, GRID_CH), lambda i: (0, 0)),
            pl.BlockSpec((1, FD // 2), lambda i: (0, 0)),
            pl.BlockSpec((1, FD), lambda i: (0, 0)),
            pl.BlockSpec((1, FD), lambda i: (0, 0)),
        ],
        out_specs=pl.BlockSpec((TPH, FD, HW), lambda i: (i, 0, 0)),
        out_shape=jax.ShapeDtypeStruct((B * T, FD, HW), jnp.float32),
        interpret=interpret,
    )(stats, interp4, grid6, wg, bg2, wr, br2, bnw2, bnb2)


def kernel(graph_data, grid_data, lat_lon_coords, graph_time_indices,
           grid_time_indices, Wg, bg, Wr, br, bn_weight, bn_bias):
    del graph_time_indices, grid_time_indices
    lat = jnp.pad(lat_lon_coords[..., 0], ((0, 0), (0, PADN - N))).reshape(-1)
    lon = jnp.pad(lat_lon_coords[..., 1], ((0, 0), (0, PADN - N))).reshape(-1)

    graph_ph = (graph_data.reshape(B, NPH, TPH, N, C)
                .transpose(0, 1, 3, 2, 4)
                .reshape(B, NPH, N, RW))

    interp_raw = _sc_scatter(lat, lon, graph_ph)
    interp4 = interp_raw.reshape(B, NPH, HW, RW)
    grid6 = grid_data.reshape(B * NPH, TPH * GRID_CH, HW)

    bg2 = bg.reshape(1, FD // 2)
    br2 = br.reshape(1, FD // 2)
    bnw2 = bn_weight.reshape(1, FD)
    bnb2 = bn_bias.reshape(1, FD)

    stats = _stats_call(interp4, grid6, Wg, bg2, Wr, br2)
    out24 = _norm_call(stats, interp4, grid6, Wg, bg2, Wr, br2, bnw2, bnb2)
    return out24.reshape(B, T, FD, 64, 64)
